# baseline (speedup 1.0000x reference)
import jax
import jax.numpy as jnp
from jax import lax
from jax.experimental import pallas as pl
from jax.experimental.pallas import tpu as pltpu
from jax.experimental.pallas import tpu_sc as plsc

B = 8
CP = 4096
N = 65536
H = 256
W = 256
K = 16
NC = 2
NS = 16
BPC = B // NC
ROWS = H // NS
CH = W
NCH = ROWS
NG = CH // 16
NBUF = 2


def _sc_body(loc_h, alp_h, cx_h, cy_h, idx_h, p0_h, px_h, py_h,
             out_h, t_loc, t_alp,
             c_idx0, c_p00, c_px0, c_py0, c_cx0, c_cy0, c_out0,
             c_idx1, c_p01, c_px1, c_py1, c_cx1, c_cy1, c_out1,
             tab_sem, in_sems, out_sems):
    c = lax.axis_index("c")
    s = lax.axis_index("s")
    bg = c * BPC
    row0 = s * ROWS

    bufs = [
        dict(idx=c_idx0, p0=c_p00, px=c_px0, py=c_py0, cx=c_cx0, cy=c_cy0,
             out=c_out0),
        dict(idx=c_idx1, p0=c_p01, px=c_px1, py=c_py1, cx=c_cx1, cy=c_cy1,
             out=c_out1),
    ]

    def in_copies(ch, buf):
        h = row0 + ch
        base_e = h * (W * K)
        base_p = h * W
        sem = in_sems.at[buf]
        bb = bufs[buf]
        return [
            pltpu.make_async_copy(idx_h.at[pl.ds(base_e, CH * K)], bb["idx"], sem),
            pltpu.make_async_copy(p0_h.at[pl.ds(base_e, CH * K)], bb["p0"], sem),
            pltpu.make_async_copy(px_h.at[pl.ds(base_e, CH * K)], bb["px"], sem),
            pltpu.make_async_copy(py_h.at[pl.ds(base_e, CH * K)], bb["py"], sem),
            pltpu.make_async_copy(cx_h.at[pl.ds(base_p, CH)], bb["cx"], sem),
            pltpu.make_async_copy(cy_h.at[pl.ds(base_p, CH)], bb["cy"], sem),
        ]

    def out_copies(ch, buf):
        h = row0 + ch
        sem = out_sems.at[buf]
        cps = []
        rbase = (h // 8) * 2048 + (h % 8) * 128
        for b in range(BPC):
            for cc in range(2):
                pbase = ((bg + b) * 2 + cc) * N + rbase
                for wt in range(2):
                    cps.append(pltpu.make_async_copy(
                        bufs[buf]["out"].at[pl.ds((b * 2 + cc) * CH + wt * 128, 128)],
                        out_h.at[pl.ds(pbase + wt * 1024, 128)], sem))
        return cps

    tab_cps = [
        pltpu.make_async_copy(loc_h.at[pl.ds(bg * CP, BPC * CP)], t_loc, tab_sem),
        pltpu.make_async_copy(alp_h.at[pl.ds(bg * CP, BPC * CP)], t_alp, tab_sem),
    ]
    for cp in tab_cps:
        cp.start()
    for buf in range(NBUF):
        for cp in in_copies(buf, buf):
            cp.start()
    for cp in tab_cps:
        cp.wait()

    zero = jnp.zeros((16,), jnp.float32)

    def chunk_iter(ch, buf):
        bb = bufs[buf]
        for cp in in_copies(ch, buf):
            cp.wait()

        @pl.when(ch >= NBUF)
        def _():
            for cp in out_copies(ch - NBUF, buf):
                cp.wait()

        @plsc.parallel_loop(0, NG)
        def group_body(g):
            cxv = bb["cx"][pl.ds(g * 16, 16)]
            cyv = bb["cy"][pl.ds(g * 16, 16)]
            gbase = (g // 8) * 1024 + (g % 8) * 16
            accs = [[zero, zero] for _ in range(BPC)]
            for k in range(K):
                koff = (k // 8) * 2048 + (k % 8) * 128
                off = gbase + koff
                idxv = bb["idx"][pl.ds(off, 16)]
                p0v = bb["p0"][pl.ds(off, 16)]
                pxv = bb["px"][pl.ds(off, 16)]
                pyv = bb["py"][pl.ds(off, 16)]
                tv = p0v - cxv * pxv - cyv * pyv
                for b in range(BPC):
                    ib = idxv + (b * CP) if b else idxv
                    wl = plsc.load_gather(t_loc, [ib])
                    wa = plsc.load_gather(t_alp, [ib])
                    lxv = plsc.bitcast(wl << 16, jnp.float32)
                    lyv = plsc.bitcast(wl, jnp.float32)
                    axv = plsc.bitcast(wa << 16, jnp.float32)
                    ayv = plsc.bitcast(wa, jnp.float32)
                    phi = tv + lxv * pxv + lyv * pyv
                    accs[b][0] = accs[b][0] + phi * axv
                    accs[b][1] = accs[b][1] + phi * ayv
            for b in range(BPC):
                for cc in range(2):
                    bb["out"][pl.ds((b * 2 + cc) * CH + g * 16, 16)] = accs[b][cc]

        for cp in out_copies(ch, buf):
            cp.start()

        @pl.when(ch + NBUF < NCH)
        def _():
            for cp in in_copies(ch + NBUF, buf):
                cp.start()

    def ring_body(g, carry):
        for buf in range(NBUF):
            chunk_iter(g * NBUF + buf, buf)
        return carry

    lax.fori_loop(0, NCH // NBUF, ring_body, 0, unroll=False)

    for buf in range(NBUF):
        for cp in out_copies(NCH - NBUF + buf, buf):
            cp.wait()


@jax.jit
def _run(loc, alp, cx, cy, idx, p0, px, py):
    mesh = plsc.VectorSubcoreMesh(core_axis_name="c", subcore_axis_name="s",
                                  num_cores=NC, num_subcores=NS)
    buf_types = [
        pltpu.VMEM((CH * K,), jnp.int32),
        pltpu.VMEM((CH * K,), jnp.float32),
        pltpu.VMEM((CH * K,), jnp.float32),
        pltpu.VMEM((CH * K,), jnp.float32),
        pltpu.VMEM((CH,), jnp.float32),
        pltpu.VMEM((CH,), jnp.float32),
        pltpu.VMEM((BPC * 2 * CH,), jnp.float32),
    ]
    f = pl.kernel(
        _sc_body,
        out_type=jax.ShapeDtypeStruct((B * 2 * N,), jnp.float32),
        mesh=mesh,
        scratch_types=[
            pltpu.VMEM((BPC * CP,), jnp.int32),
            pltpu.VMEM((BPC * CP,), jnp.int32),
            *buf_types,
            *buf_types,
            pltpu.SemaphoreType.DMA,
            pltpu.SemaphoreType.DMA((NBUF,)),
            pltpu.SemaphoreType.DMA((NBUF,)),
        ],
        compiler_params=pltpu.CompilerParams(needs_layout_passes=False),
    )
    return f(loc, alp, cx, cy, idx, p0, px, py)


def _phys_view(a):
    t = a.reshape(H, 2, 128, K // 8, 8)
    return t.transpose(0, 3, 1, 4, 2).reshape(-1)


def _pack2(x, y):
    xb = lax.bitcast_convert_type(x.astype(jnp.bfloat16), jnp.uint16)
    yb = lax.bitcast_convert_type((y * (1.0 - 2.0 ** -8)).astype(jnp.bfloat16),
                                  jnp.uint16)
    w = (yb.astype(jnp.uint32) << 16) | xb.astype(jnp.uint32)
    return lax.bitcast_convert_type(w, jnp.int32)


def kernel(cpoint_loc, alpha, cpoints_0, select_index, phi_0, phi_x, phi_y):
    loc = _pack2(cpoint_loc[:, :, 0], cpoint_loc[:, :, 1]).reshape(-1)
    alp = _pack2(alpha[:, :, 0], alpha[:, :, 1]).reshape(-1)
    cx = cpoints_0[..., 0].reshape(-1)
    cy = cpoints_0[..., 1].reshape(-1)
    idx = _phys_view(select_index)
    p0 = _phys_view(phi_0)
    px = _phys_view(phi_x)
    py = _phys_view(phi_y)
    out = _run(loc, alp, cx, cy, idx, p0, px, py)
    o = out.reshape(B, 2, H // 8, 2, 8, 128)
    return o.transpose(0, 1, 2, 4, 3, 5).reshape(B, 2, H, W)


# speedup vs baseline: 2.7636x; 2.7636x over previous
import jax
import jax.numpy as jnp
from jax import lax
from jax.experimental import pallas as pl
from jax.experimental.pallas import tpu as pltpu
from jax.experimental.pallas import tpu_sc as plsc

B = 8
CP = 4096
N = 65536
H = 256
W = 256
K = 16
NC = 2
NS = 16
BPC = B // NC
ROWS = H // NS
CH = W
NCH = ROWS
NG = CH // 16
NBUF = 2


def _sc_body(loc_h, alp_h, cx_h, cy_h, idx_h, p0_h, px_h, py_h,
             out_h, t_loc, t_alp,
             c_idx0, c_p00, c_px0, c_py0, c_cx0, c_cy0, c_out0,
             c_idx1, c_p01, c_px1, c_py1, c_cx1, c_cy1, c_out1,
             tab_sem, in_sems, out_sems):
    c = lax.axis_index("c")
    s = lax.axis_index("s")
    bg = c * BPC
    row0 = s * ROWS

    bufs = [
        dict(idx=c_idx0, p0=c_p00, px=c_px0, py=c_py0, cx=c_cx0, cy=c_cy0,
             out=c_out0),
        dict(idx=c_idx1, p0=c_p01, px=c_px1, py=c_py1, cx=c_cx1, cy=c_cy1,
             out=c_out1),
    ]

    def in_copies(ch, buf):
        h = row0 + ch
        base_e = h * (W * K)
        base_p = h * W
        sem = in_sems.at[buf]
        bb = bufs[buf]
        return [
            pltpu.make_async_copy(idx_h.at[pl.ds(base_e, CH * K)], bb["idx"], sem),
            pltpu.make_async_copy(p0_h.at[pl.ds(base_e, CH * K)], bb["p0"], sem),
            pltpu.make_async_copy(px_h.at[pl.ds(base_e, CH * K)], bb["px"], sem),
            pltpu.make_async_copy(py_h.at[pl.ds(base_e, CH * K)], bb["py"], sem),
            pltpu.make_async_copy(cx_h.at[pl.ds(base_p, CH)], bb["cx"], sem),
            pltpu.make_async_copy(cy_h.at[pl.ds(base_p, CH)], bb["cy"], sem),
        ]

    def out_copies(ch, buf):
        h = row0 + ch
        sem = out_sems.at[buf]
        cps = []
        rbase = (h // 8) * 2048 + (h % 8) * 128
        for b in range(BPC):
            for cc in range(2):
                pbase = ((bg + b) * 2 + cc) * N + rbase
                for wt in range(2):
                    cps.append(pltpu.make_async_copy(
                        bufs[buf]["out"].at[pl.ds((b * 2 + cc) * CH + wt * 128, 128)],
                        out_h.at[pl.ds(pbase + wt * 1024, 128)], sem))
        return cps

    tab_cps = [
        pltpu.make_async_copy(loc_h.at[pl.ds(bg * CP, BPC * CP)], t_loc, tab_sem),
        pltpu.make_async_copy(alp_h.at[pl.ds(bg * CP, BPC * CP)], t_alp, tab_sem),
    ]
    for cp in tab_cps:
        cp.start()
    for buf in range(NBUF):
        for cp in in_copies(buf, buf):
            cp.start()
    for cp in tab_cps:
        cp.wait()

    zero = jnp.zeros((16,), jnp.float32)

    def chunk_iter(ch, buf):
        bb = bufs[buf]
        for cp in in_copies(ch, buf):
            cp.wait()

        @pl.when(ch >= NBUF)
        def _():
            for cp in out_copies(ch - NBUF, buf):
                cp.wait()

        def group_body(g, gcarry):
            cxv = bb["cx"][pl.ds(g * 16, 16)]
            cyv = bb["cy"][pl.ds(g * 16, 16)]
            gbase = (g // 8) * 1024 + (g % 8) * 16
            acc0 = [[zero, zero] for _ in range(BPC)]

            def k_half(kt, kaccs):
                accs = [[kaccs[2 * b], kaccs[2 * b + 1]] for b in range(BPC)]
                for kk in range(K // 2):
                    off = gbase + kt * 2048 + kk * 128
                    idxv = bb["idx"][pl.ds(off, 16)]
                    p0v = bb["p0"][pl.ds(off, 16)]
                    pxv = bb["px"][pl.ds(off, 16)]
                    pyv = bb["py"][pl.ds(off, 16)]
                    tv = p0v - cxv * pxv - cyv * pyv
                    for b in range(BPC):
                        ib = idxv + (b * CP) if b else idxv
                        wl = plsc.load_gather(t_loc, [ib])
                        wa = plsc.load_gather(t_alp, [ib])
                        lxv = plsc.bitcast(wl << 16, jnp.float32)
                        lyv = plsc.bitcast(wl, jnp.float32)
                        axv = plsc.bitcast(wa << 16, jnp.float32)
                        ayv = plsc.bitcast(wa, jnp.float32)
                        phi = tv + lxv * pxv + lyv * pyv
                        accs[b][0] = accs[b][0] + phi * axv
                        accs[b][1] = accs[b][1] + phi * ayv
                return [a for bb2 in accs for a in bb2]

            kaccs = lax.fori_loop(0, 2, k_half,
                                  [a for bb2 in acc0 for a in bb2])
            accs = [[kaccs[2 * b], kaccs[2 * b + 1]] for b in range(BPC)]
            for b in range(BPC):
                for cc in range(2):
                    bb["out"][pl.ds((b * 2 + cc) * CH + g * 16, 16)] = accs[b][cc]
            return gcarry

        lax.fori_loop(0, NG, group_body, 0)

        for cp in out_copies(ch, buf):
            cp.start()

        @pl.when(ch + NBUF < NCH)
        def _():
            for cp in in_copies(ch + NBUF, buf):
                cp.start()

    def ring_body(g, carry):
        for buf in range(NBUF):
            chunk_iter(g * NBUF + buf, buf)
        return carry

    lax.fori_loop(0, NCH // NBUF, ring_body, 0, unroll=False)

    for buf in range(NBUF):
        for cp in out_copies(NCH - NBUF + buf, buf):
            cp.wait()


@jax.jit
def _run(loc, alp, cx, cy, idx, p0, px, py):
    mesh = plsc.VectorSubcoreMesh(core_axis_name="c", subcore_axis_name="s",
                                  num_cores=NC, num_subcores=NS)
    buf_types = [
        pltpu.VMEM((CH * K,), jnp.int32),
        pltpu.VMEM((CH * K,), jnp.float32),
        pltpu.VMEM((CH * K,), jnp.float32),
        pltpu.VMEM((CH * K,), jnp.float32),
        pltpu.VMEM((CH,), jnp.float32),
        pltpu.VMEM((CH,), jnp.float32),
        pltpu.VMEM((BPC * 2 * CH,), jnp.float32),
    ]
    f = pl.kernel(
        _sc_body,
        out_type=jax.ShapeDtypeStruct((B * 2 * N,), jnp.float32),
        mesh=mesh,
        scratch_types=[
            pltpu.VMEM((BPC * CP,), jnp.int32),
            pltpu.VMEM((BPC * CP,), jnp.int32),
            *buf_types,
            *buf_types,
            pltpu.SemaphoreType.DMA,
            pltpu.SemaphoreType.DMA((NBUF,)),
            pltpu.SemaphoreType.DMA((NBUF,)),
        ],
        compiler_params=pltpu.CompilerParams(needs_layout_passes=False),
    )
    return f(loc, alp, cx, cy, idx, p0, px, py)


def _phys_view(a):
    t = a.reshape(H, 2, 128, K // 8, 8)
    return t.transpose(0, 3, 1, 4, 2).reshape(-1)


def _pack2(x, y):
    xb = lax.bitcast_convert_type(x.astype(jnp.bfloat16), jnp.uint16)
    yb = lax.bitcast_convert_type((y * (1.0 - 2.0 ** -8)).astype(jnp.bfloat16),
                                  jnp.uint16)
    w = (yb.astype(jnp.uint32) << 16) | xb.astype(jnp.uint32)
    return lax.bitcast_convert_type(w, jnp.int32)


def kernel(cpoint_loc, alpha, cpoints_0, select_index, phi_0, phi_x, phi_y):
    loc = _pack2(cpoint_loc[:, :, 0], cpoint_loc[:, :, 1]).reshape(-1)
    alp = _pack2(alpha[:, :, 0], alpha[:, :, 1]).reshape(-1)
    cx = cpoints_0[..., 0].reshape(-1)
    cy = cpoints_0[..., 1].reshape(-1)
    idx = _phys_view(select_index)
    p0 = _phys_view(phi_0)
    px = _phys_view(phi_x)
    py = _phys_view(phi_y)
    out = _run(loc, alp, cx, cy, idx, p0, px, py)
    o = out.reshape(B, 2, H // 8, 2, 8, 128)
    return o.transpose(0, 1, 2, 4, 3, 5).reshape(B, 2, H, W)


# speedup vs baseline: 2.8014x; 1.0137x over previous
import jax
import jax.numpy as jnp
from jax import lax
from jax.experimental import pallas as pl
from jax.experimental.pallas import tpu as pltpu
from jax.experimental.pallas import tpu_sc as plsc

B = 8
CP = 4096
N = 65536
H = 256
W = 256
K = 16
NC = 2
NS = 16
BPC = B // NC
ROWS = H // NS
CH = W
NCH = ROWS
NG = CH // 16
NBUF = 2


def _sc_body(loc_h, alp_h, cxy_h, idx_h, p0_h, px_h, py_h,
             out_h, t_loc, t_alp,
             c_idx0, c_p00, c_px0, c_py0, c_cx0, c_cy0, c_out0,
             c_idx1, c_p01, c_px1, c_py1, c_cx1, c_cy1, c_out1,
             tab_sem, in_sems, out_sems):
    c = lax.axis_index("c")
    s = lax.axis_index("s")
    bg = c * BPC
    row0 = s * ROWS

    bufs = [
        dict(idx=c_idx0, p0=c_p00, px=c_px0, py=c_py0, cx=c_cx0, cy=c_cy0,
             out=c_out0),
        dict(idx=c_idx1, p0=c_p01, px=c_px1, py=c_py1, cx=c_cx1, cy=c_cy1,
             out=c_out1),
    ]

    def in_copies(ch, buf):
        h = row0 + ch
        base_e = h * (W * K)
        base_p = h * W
        sem = in_sems.at[buf]
        bb = bufs[buf]
        return [
            pltpu.make_async_copy(idx_h.at[pl.ds(base_e, CH * K)], bb["idx"], sem),
            pltpu.make_async_copy(p0_h.at[pl.ds(base_e, CH * K)], bb["p0"], sem),
            pltpu.make_async_copy(px_h.at[pl.ds(base_e, CH * K)], bb["px"], sem),
            pltpu.make_async_copy(py_h.at[pl.ds(base_e, CH * K)], bb["py"], sem),
            pltpu.make_async_copy(cxy_h.at[pl.ds((2 * h) * W, CH)], bb["cx"], sem),
            pltpu.make_async_copy(cxy_h.at[pl.ds((2 * h + 1) * W, CH)], bb["cy"], sem),
        ]

    def out_copies(ch, buf):
        h = row0 + ch
        sem = out_sems.at[buf]
        cps = []
        rbase = (h // 8) * 2048 + (h % 8) * 128
        for b in range(BPC):
            for cc in range(2):
                pbase = ((bg + b) * 2 + cc) * N + rbase
                for wt in range(2):
                    cps.append(pltpu.make_async_copy(
                        bufs[buf]["out"].at[pl.ds((b * 2 + cc) * CH + wt * 128, 128)],
                        out_h.at[pl.ds(pbase + wt * 1024, 128)], sem))
        return cps

    tab_cps = [
        pltpu.make_async_copy(loc_h.at[pl.ds(bg * CP, BPC * CP)], t_loc, tab_sem),
        pltpu.make_async_copy(alp_h.at[pl.ds(bg * CP, BPC * CP)], t_alp, tab_sem),
    ]
    for cp in tab_cps:
        cp.start()
    for buf in range(NBUF):
        for cp in in_copies(buf, buf):
            cp.start()
    for cp in tab_cps:
        cp.wait()

    zero = jnp.zeros((16,), jnp.float32)

    def chunk_iter(ch, buf):
        bb = bufs[buf]
        for cp in in_copies(ch, buf):
            cp.wait()

        @pl.when(ch >= NBUF)
        def _():
            for cp in out_copies(ch - NBUF, buf):
                cp.wait()

        def group_body(g, gcarry):
            cxv = bb["cx"][pl.ds(g * 16, 16)]
            cyv = bb["cy"][pl.ds(g * 16, 16)]
            gbase = (g // 8) * 1024 + (g % 8) * 16
            accs = [[zero, zero] for _ in range(BPC)]
            for k in range(K):
                koff = (k // 8) * 2048 + (k % 8) * 128
                off = gbase + koff
                idxv = bb["idx"][pl.ds(off, 16)]
                p0v = bb["p0"][pl.ds(off, 16)]
                pxv = bb["px"][pl.ds(off, 16)]
                pyv = bb["py"][pl.ds(off, 16)]
                tv = p0v - cxv * pxv - cyv * pyv
                for b in range(BPC):
                    ib = idxv + (b * CP) if b else idxv
                    wl = plsc.load_gather(t_loc, [ib])
                    wa = plsc.load_gather(t_alp, [ib])
                    lxv = plsc.bitcast(wl << 16, jnp.float32)
                    lyv = plsc.bitcast(wl, jnp.float32)
                    axv = plsc.bitcast(wa << 16, jnp.float32)
                    ayv = plsc.bitcast(wa, jnp.float32)
                    phi = tv + lxv * pxv + lyv * pyv
                    accs[b][0] = accs[b][0] + phi * axv
                    accs[b][1] = accs[b][1] + phi * ayv
            for b in range(BPC):
                for cc in range(2):
                    bb["out"][pl.ds((b * 2 + cc) * CH + g * 16, 16)] = accs[b][cc]
            return gcarry

        lax.fori_loop(0, NG, group_body, 0)

        for cp in out_copies(ch, buf):
            cp.start()

        @pl.when(ch + NBUF < NCH)
        def _():
            for cp in in_copies(ch + NBUF, buf):
                cp.start()

    def ring_body(g, carry):
        for buf in range(NBUF):
            chunk_iter(g * NBUF + buf, buf)
        return carry

    lax.fori_loop(0, NCH // NBUF, ring_body, 0, unroll=False)

    for buf in range(NBUF):
        for cp in out_copies(NCH - NBUF + buf, buf):
            cp.wait()


@jax.jit
def _run(loc, alp, cxy, idx, p0, px, py):
    mesh = plsc.VectorSubcoreMesh(core_axis_name="c", subcore_axis_name="s",
                                  num_cores=NC, num_subcores=NS)
    buf_types = [
        pltpu.VMEM((CH * K,), jnp.int32),
        pltpu.VMEM((CH * K,), jnp.float32),
        pltpu.VMEM((CH * K,), jnp.float32),
        pltpu.VMEM((CH * K,), jnp.float32),
        pltpu.VMEM((CH,), jnp.float32),
        pltpu.VMEM((CH,), jnp.float32),
        pltpu.VMEM((BPC * 2 * CH,), jnp.float32),
    ]
    f = pl.kernel(
        _sc_body,
        out_type=jax.ShapeDtypeStruct((B * 2 * N,), jnp.float32),
        mesh=mesh,
        scratch_types=[
            pltpu.VMEM((BPC * CP,), jnp.int32),
            pltpu.VMEM((BPC * CP,), jnp.int32),
            *buf_types,
            *buf_types,
            pltpu.SemaphoreType.DMA,
            pltpu.SemaphoreType.DMA((NBUF,)),
            pltpu.SemaphoreType.DMA((NBUF,)),
        ],
        compiler_params=pltpu.CompilerParams(needs_layout_passes=False),
    )
    return f(loc, alp, cxy, idx, p0, px, py)


def _phys_view(a):
    t = a.reshape(H, 2, 128, K // 8, 8)
    return t.transpose(0, 3, 1, 4, 2).reshape(-1)


def _pack2(x, y):
    def rnd16(f):
        bits = lax.bitcast_convert_type(f, jnp.uint32)
        return (bits + 0x7FFF + ((bits >> 16) & 1)) >> 16

    xb = rnd16(x)
    yb = rnd16(y * (1.0 - 2.0 ** -8))
    w = (yb << 16) | xb
    return lax.bitcast_convert_type(w, jnp.int32)


def kernel(cpoint_loc, alpha, cpoints_0, select_index, phi_0, phi_x, phi_y):
    loc = _pack2(cpoint_loc[:, :, 0], cpoint_loc[:, :, 1]).reshape(-1)
    alp = _pack2(alpha[:, :, 0], alpha[:, :, 1]).reshape(-1)
    cxy = cpoints_0.reshape(H, W, 2).transpose(0, 2, 1).reshape(-1)
    idx = _phys_view(select_index)
    p0 = _phys_view(phi_0)
    px = _phys_view(phi_x)
    py = _phys_view(phi_y)
    out = _run(loc, alp, cxy, idx, p0, px, py)
    o = out.reshape(B, 2, H // 8, 2, 8, 128)
    return o.transpose(0, 1, 2, 4, 3, 5).reshape(B, 2, H, W)
